# baseline (speedup 1.0000x reference)
import functools

import jax
import jax.numpy as jnp
import numpy as np
from jax import lax
from jax.experimental import pallas as pl
from jax.experimental.pallas import tpu as pltpu
from jax.experimental.pallas import tpu_sc as plsc

F32 = jnp.float32
I32 = jnp.int32

NC, NS, NW = 2, 16, 32
N = 10000
Q4 = N // 4
E1, E2, E3 = 80000, 10000, 70000
E1P, E2P, E3P = 80640, 10240, 70400
CB1, CB2, CB3 = E1P // NW, E2P // NW, E3P // NW
CBS1, CBS3 = E1P // NS, E3P // NS
CBG2, CBSC2 = E2P // NS, E2P // NS
ACCR = 2504
ACCS = 2560
RPW = 79
OUTR = RPW * NW
BATCH = 512

L = 4
SZ = 7
AXIS_LEN = L * (1 + 2 * SZ)


def _mesh():
    return plsc.VectorSubcoreMesh(core_axis_name="c", subcore_axis_name="s")


_SC_PARAMS = pltpu.CompilerParams(use_tc_tiling_on_sc=False,
                                  needs_layout_passes=False)


def _make_sc_gather(specs):
    n = len(specs)
    cbmax = max(cb for cb, _ in specs)
    wmax = max(w for _, w in specs)
    out_type = tuple(jax.ShapeDtypeStruct((cb * NW, w), F32) for cb, w in specs)
    if n == 1:
        out_type = out_type[0]
    scratch = [pltpu.VMEM((cbmax,), I32), pltpu.VMEM((cbmax, wmax), F32),
               pltpu.SemaphoreType.DMA]

    @functools.partial(pl.kernel, mesh=_mesh(), out_type=out_type,
                       scratch_types=scratch, compiler_params=_SC_PARAMS)
    def k(*refs):
        tables = refs[:n]
        idxs = refs[n:2 * n]
        outs = refs[2 * n:3 * n]
        idx_v, rows_v, sem = refs[3 * n:3 * n + 3]
        wid = lax.axis_index("s") * NC + lax.axis_index("c")
        for g, (cb, w) in enumerate(specs):
            base = wid * cb
            iv = idx_v.at[pl.ds(0, cb)]
            rv = rows_v.at[pl.ds(0, cb), pl.ds(0, w)]
            pltpu.sync_copy(idxs[g].at[pl.ds(base, cb)], iv)
            pltpu.async_copy(tables[g].at[iv], rv, sem).wait()
            pltpu.sync_copy(rv, outs[g].at[pl.ds(base, cb)])

    return k


def _make_sc_scatter_gather(cbs, cbg, side_specs):
    cbmax = max([cbs] + list(side_specs))
    out_type = (jax.ShapeDtypeStruct((ACCR, 16), F32),
                jax.ShapeDtypeStruct((cbg * NS, 16), F32)) + tuple(
        jax.ShapeDtypeStruct((cb * NS, 16), F32) for cb in side_specs)
    nside = len(side_specs)
    scratch = [pltpu.VMEM((cbmax, 16), F32), pltpu.VMEM((cbmax,), I32),
               pltpu.VMEM((cbg,), I32), pltpu.VMEM((cbg, 16), F32),
               pltpu.VMEM_SHARED((ACCR, 16), F32), pltpu.SemaphoreType.DMA]

    @functools.partial(pl.kernel, mesh=_mesh(), out_type=out_type,
                       scratch_types=scratch, compiler_params=_SC_PARAMS)
    def k(*refs):
        (vals_hbm, dstl_hbm, gidx_hbm, zeros_hbm, table_hbm) = refs[:5]
        sidx_hbm = refs[5:5 + nside]
        acc_out, g_out = refs[5 + nside], refs[6 + nside]
        souts = refs[7 + nside:7 + 2 * nside]
        vals_v, idx_v, gi_v, gr_v, acc_sh, sem = refs[7 + 2 * nside:]
        cid = lax.axis_index("c")
        sid = lax.axis_index("s")

        @pl.when(cid == 0)
        def _():
            @pl.when(sid == 0)
            def _():
                pltpu.sync_copy(zeros_hbm, acc_sh)

            plsc.subcore_barrier()
            base = sid * cbs
            pltpu.sync_copy(vals_hbm.at[pl.ds(base, cbs)],
                            vals_v.at[pl.ds(0, cbs), :])
            pltpu.sync_copy(dstl_hbm.at[pl.ds(base, cbs)],
                            idx_v.at[pl.ds(0, cbs)])
            pltpu.sync_copy(vals_v.at[pl.ds(0, cbs), :],
                            acc_sh.at[idx_v.at[pl.ds(0, cbs)]], add=True)
            plsc.subcore_barrier()
            gbase = sid * cbg
            pltpu.sync_copy(gidx_hbm.at[pl.ds(gbase, cbg)], gi_v)
            pltpu.async_copy(acc_sh.at[gi_v], gr_v, sem).wait()
            pltpu.sync_copy(gr_v, g_out.at[pl.ds(gbase, cbg)])

            @pl.when(sid == 0)
            def _():
                pltpu.sync_copy(acc_sh, acc_out)

        @pl.when(cid == 1)
        def _():
            for g in range(nside):
                cb = side_specs[g]
                gb = sid * cb
                iv = idx_v.at[pl.ds(0, cb)]
                rv = vals_v.at[pl.ds(0, cb), :]
                pltpu.sync_copy(sidx_hbm[g].at[pl.ds(gb, cb)], iv)
                pltpu.async_copy(table_hbm.at[iv], rv, sem).wait()
                pltpu.sync_copy(rv, souts[g].at[pl.ds(gb, cb)])

    return k


def _make_sc_scatter_single(cbs):
    scratch = [pltpu.VMEM((cbs, 16), F32), pltpu.VMEM((cbs,), I32),
               pltpu.VMEM_SHARED((ACCR, 16), F32)]

    @functools.partial(pl.kernel, mesh=_mesh(),
                       out_type=jax.ShapeDtypeStruct((ACCR, 16), F32),
                       scratch_types=scratch, compiler_params=_SC_PARAMS)
    def k(vals_hbm, dstl_hbm, zeros_hbm, acc_out, vals_v, idx_v, acc_sh):
        cid = lax.axis_index("c")
        sid = lax.axis_index("s")

        @pl.when(cid == 0)
        def _():
            @pl.when(sid == 0)
            def _():
                pltpu.sync_copy(zeros_hbm, acc_sh)

            plsc.subcore_barrier()
            base = sid * cbs
            pltpu.sync_copy(vals_hbm.at[pl.ds(base, cbs)], vals_v)
            pltpu.sync_copy(dstl_hbm.at[pl.ds(base, cbs)], idx_v)
            pltpu.sync_copy(vals_v, acc_sh.at[idx_v], add=True)
            plsc.subcore_barrier()

            @pl.when(sid == 0)
            def _():
                pltpu.sync_copy(acc_sh, acc_out)

    return k


def _make_sc_reduce2():
    EH = E2P // 2
    CBH = EH // NS
    NVEC = EH // 16
    RPW2 = OUTR // NS
    scratch = [
        pltpu.VMEM((EH,), I32),
        pltpu.VMEM((EH + BATCH + 16,), I32),
        pltpu.VMEM((EH + 16,), I32),
        pltpu.VMEM((BATCH, 32), F32),
        pltpu.VMEM((RPW2, 32), F32),
        pltpu.VMEM((CBH, 48), F32),
        pltpu.VMEM((CBH,), I32),
        pltpu.VMEM_SHARED((ACCS, 48), F32),
        pltpu.SemaphoreType.DMA,
    ]

    @functools.partial(pl.kernel, mesh=_mesh(),
                       out_type=(jax.ShapeDtypeStruct((NC * ACCS, 48), F32),
                                 jax.ShapeDtypeStruct((NC * OUTR, 32), F32)),
                       scratch_types=scratch, compiler_params=_SC_PARAMS)
    def k(efc1_hbm, efc2_hbm, dstl_hbm, zeros_hbm, sum_out, max_out,
          dst_v, hit_ids, hit_dst, rows_v, acc, svals, sidx, acc_sh, sem):
        cid = lax.axis_index("c")
        sid = lax.axis_index("s")
        ebase = cid * EH

        @pl.when(sid == 0)
        def _():
            pltpu.sync_copy(zeros_hbm, acc_sh)

        plsc.subcore_barrier()
        base = ebase + sid * CBH
        pltpu.sync_copy(efc1_hbm.at[pl.ds(base, CBH)], svals)
        pltpu.sync_copy(dstl_hbm.at[pl.ds(base, CBH)], sidx)
        pltpu.sync_copy(svals, acc_sh.at[sidx], add=True)
        plsc.subcore_barrier()

        @pl.when(sid == 0)
        def _():
            pltpu.sync_copy(acc_sh, sum_out.at[pl.ds(cid * ACCS, ACCS)])

        lo = sid * RPW2
        hi = lo + RPW2
        pltpu.sync_copy(dstl_hbm.at[pl.ds(ebase, EH)], dst_v)

        ninf = jnp.full((16,), -3.4e38, F32)
        zi = jnp.zeros((16,), I32)

        def init_r(r, c):
            acc[r, pl.ds(0, 16)] = ninf
            acc[r, pl.ds(16, 16)] = ninf
            return c

        lax.fori_loop(0, RPW2, init_r, 0)

        iota16 = lax.iota(I32, 16)

        def scan_b(v, off_v):
            dv = dst_v[pl.ds(v * 16, 16)]
            m = (dv >= lo) & (dv < hi)
            pc = plsc.cumsum(m.astype(I32))
            pos = off_v + pc - 1
            plsc.store_scatter(hit_ids, [pos], ebase + iota16 + v * 16, mask=m)
            plsc.store_scatter(hit_dst, [pos], dv, mask=m)
            return off_v + plsc.all_reduce_population_count(m)

        off_v = lax.fori_loop(0, NVEC, scan_b, jnp.zeros((16,), I32))
        nh = off_v[0]

        def zt(i, c):
            hit_ids[pl.ds(nh + i * 16, 16)] = zi
            return c

        lax.fori_loop(0, BATCH // 16 + 1, zt, 0)

        nbt = (nh + BATCH - 1) // BATCH

        def batch_b(b, c):
            ids = hit_ids.at[pl.ds(b * BATCH, BATCH)]
            pltpu.async_copy(efc2_hbm.at[ids], rows_v, sem).wait()
            nb = jnp.minimum(BATCH, nh - b * BATCH)

            def e_b(j, cc):
                dvec = hit_dst[pl.ds(b * BATCH + j, 16)]
                r = dvec[0] - lo
                acc[r, pl.ds(0, 16)] = jnp.maximum(acc[r, pl.ds(0, 16)],
                                                   rows_v[j, pl.ds(0, 16)])
                acc[r, pl.ds(16, 16)] = jnp.maximum(acc[r, pl.ds(16, 16)],
                                                    rows_v[j, pl.ds(16, 16)])
                return cc

            lax.fori_loop(0, nb, e_b, 0)
            return c

        lax.fori_loop(0, nbt, batch_b, 0)
        pltpu.sync_copy(acc, max_out.at[pl.ds(cid * OUTR + lo, RPW2)])

    return k


def _run_net_mlp(parts, w1_parts, b1, mids, wlast, blast, res_mask, be):
    ep = parts[0].shape[0]
    nblk = ep // be
    npart = len(parts)

    def body(*refs):
        ins = refs[:npart]
        i = npart
        w1r = refs[i:i + npart]; i += npart
        b1r = refs[i]; i += 1
        midr = refs[i:i + 6]; i += 6
        wlr, blr, mr = refs[i], refs[i + 1], refs[i + 2]
        out = refs[-1]
        x0 = ins[0][...]
        s = jnp.dot(x0, w1r[0][...], preferred_element_type=F32)
        for p in range(1, npart):
            s = s + jnp.dot(ins[p][...], w1r[p][...], preferred_element_type=F32)
        h = jax.nn.relu(s + b1r[...])
        for m in range(3):
            h = jax.nn.relu(jnp.dot(h, midr[2 * m][...],
                                    preferred_element_type=F32) + midr[2 * m + 1][...])
        y = jnp.dot(h, wlr[...], preferred_element_type=F32) + blr[...]
        out[...] = y + x0 * mr[...]

    in_specs = [pl.BlockSpec((be, p.shape[1]), lambda i: (i, 0)) for p in parts]
    wargs = list(w1_parts) + [b1]
    for (w, b) in mids:
        wargs += [w, b]
    wargs += [wlast, blast, res_mask]
    in_specs += [pl.BlockSpec(w.shape, lambda i: (0, 0)) for w in wargs]
    return pl.pallas_call(
        body,
        grid=(nblk,),
        in_specs=in_specs,
        out_specs=pl.BlockSpec((be, wlast.shape[1]), lambda i: (i, 0)),
        out_shape=jax.ShapeDtypeStruct((ep, wlast.shape[1]), F32),
    )(*parts, *wargs)


def _run_level2(g_l2, g_s2, g_d2, efp, lq_w, ca_w, be):
    nblk = E2P // be
    lq_flat = [lq_w[0], lq_w[1], lq_w[2], lq_w[3]]
    for (w, b) in lq_w[4]:
        lq_flat += [w, b]
    lq_flat += [lq_w[5], lq_w[6]]
    ca_flat = [ca_w[0], ca_w[1], ca_w[2], ca_w[3], ca_w[4]]
    for (w, b) in ca_w[5]:
        ca_flat += [w, b]
    ca_flat += [ca_w[6], ca_w[7]]
    nlq = len(lq_flat)

    def body(*refs):
        gl, gs, gd, eft = (r[...] for r in refs[:4])
        i = 4
        lw = refs[i:i + nlq]; i += nlq
        cw = refs[i:i + len(ca_flat)]; i += len(ca_flat)
        o1c, o2, oce = refs[-3], refs[-2], refs[-1]

        def mm(a, b):
            return jnp.dot(a, b, preferred_element_type=F32)

        h = jax.nn.relu(mm(gl, lw[0][...]) + mm(gs, lw[1][...]) +
                        mm(gd, lw[2][...]) + lw[3][...])
        for m in range(3):
            h = jax.nn.relu(mm(h, lw[4 + 2 * m][...]) + lw[5 + 2 * m][...])
        s = mm(h, lw[10][...]) + lw[11][...]
        st = jnp.transpose(s)

        trows = []
        for l in range(L):
            k1t = eft[15 * l + 1:15 * l + 8, :]
            k2t = eft[15 * l + 8:15 * l + 15, :]
            tabt = eft[AXIS_LEN + 49 * l:AXIS_LEN + 49 * l + 49, :]
            kmt = jnp.concatenate([k1t[i2:i2 + 1, :] * k2t for i2 in range(SZ)],
                                  axis=0)
            for hh in range(4):
                lg = st[4 * l + hh:4 * l + hh + 1, :] * kmt
                mx = jnp.max(lg, axis=0, keepdims=True)
                p = jnp.exp(lg - mx)
                z = jnp.sum(p, axis=0, keepdims=True)
                trows.append(jnp.sum(p * tabt, axis=0, keepdims=True) / z)
        t = jnp.transpose(jnp.concatenate(trows, axis=0))

        h2 = jax.nn.relu(mm(gl, cw[0][...]) + mm(gs, cw[1][...]) +
                         mm(gd, cw[2][...]) + mm(t, cw[3][...]) + cw[4][...])
        for m in range(3):
            h2 = jax.nn.relu(mm(h2, cw[5 + 2 * m][...]) + cw[6 + 2 * m][...])
        x80 = mm(h2, cw[11][...]) + cw[12][...]
        kk = jax.nn.sigmoid(x80[:, 0:1])
        ones = jnp.ones((x80.shape[0], 1), F32)
        zer = jnp.zeros((x80.shape[0], 15), F32)
        o1c[...] = jnp.concatenate([x80[:, 1:33] * kk, ones, zer], axis=1)
        o2[...] = x80[:, 33:65] * kk
        oce[...] = x80[:, 65:69]

    in_specs = [
        pl.BlockSpec((be, 16), lambda i: (i, 0)),
        pl.BlockSpec((be, 16), lambda i: (i, 0)),
        pl.BlockSpec((be, 16), lambda i: (i, 0)),
        pl.BlockSpec((256, be), lambda i: (0, i)),
    ]
    wargs = lq_flat + ca_flat
    in_specs += [pl.BlockSpec(w.shape, lambda i: (0, 0)) for w in wargs]
    return pl.pallas_call(
        body,
        grid=(nblk,),
        in_specs=in_specs,
        out_specs=[pl.BlockSpec((be, 48), lambda i: (i, 0)),
                   pl.BlockSpec((be, 32), lambda i: (i, 0)),
                   pl.BlockSpec((be, 4), lambda i: (i, 0))],
        out_shape=[jax.ShapeDtypeStruct((E2P, 48), F32),
                   jax.ShapeDtypeStruct((E2P, 32), F32),
                   jax.ShapeDtypeStruct((E2P, 4), F32)],
    )(g_l2, g_s2, g_d2, efp, *wargs)


def _run_cellreduce(nfq3, s0, s1, m0, m1, wa, wb, wc, b1, mids, wlast, blast,
                    be):
    ep = nfq3.shape[0]
    nblk = ep // be
    wargs = [wa, wb, wc, b1]
    for (w, b) in mids:
        wargs += [w, b]
    wargs += [wlast, blast]

    def body(*refs):
        xn, xs0, xs1, xm0, xm1 = (r[...] for r in refs[:5])
        w = refs[5:-1]
        out = refs[-1]
        xs = xs0 + xs1
        xm = jnp.maximum(xm0, xm1)
        cnt = xs[:, 32:33]
        fixed = jnp.where(((xm - xm) == 0.0) & (cnt > 0.0), xm, 0.0)
        h = jax.nn.relu(jnp.dot(xn, w[0][...], preferred_element_type=F32) +
                        jnp.dot(xs, w[1][...], preferred_element_type=F32) +
                        jnp.dot(fixed, w[2][...], preferred_element_type=F32) +
                        w[3][...])
        for m in range(3):
            h = jax.nn.relu(jnp.dot(h, w[4 + 2 * m][...],
                                    preferred_element_type=F32) + w[5 + 2 * m][...])
        out[...] = jnp.dot(h, w[10][...], preferred_element_type=F32) + w[11][...]

    in_specs = [pl.BlockSpec((be, 16), lambda i: (i, 0)),
                pl.BlockSpec((be, 48), lambda i: (i, 0)),
                pl.BlockSpec((be, 48), lambda i: (i, 0)),
                pl.BlockSpec((be, 32), lambda i: (i, 0)),
                pl.BlockSpec((be, 32), lambda i: (i, 0))]
    in_specs += [pl.BlockSpec(w.shape, lambda i: (0, 0)) for w in wargs]
    return pl.pallas_call(
        body,
        grid=(nblk,),
        in_specs=in_specs,
        out_specs=pl.BlockSpec((be, 16), lambda i: (i, 0)),
        out_shape=jax.ShapeDtypeStruct((ep, 16), F32),
    )(nfq3, s0, s1, m0, m1, *wargs)


def _fold_weights(params):
    npp = params['netprop']
    lq = params['lut_query']
    ca = params['cellarc']
    cr = params['cellreduce']
    msa = params['msa']

    def bias(b):
        return b.reshape(1, -1)

    W1 = npp[0][0]
    w1_src1 = jnp.zeros((16, 64), F32).at[0:4].set(W1[:, 0:4].T).at[4:14].set(W1[:, 4:14].T)
    w1_dst = jnp.zeros((16, 64), F32).at[0:10].set(W1[:, 14:24].T)
    w1_l3 = jnp.zeros((16, 64), F32).at[0:4].set(W1[:, 0:4].T)
    w1_s3 = jnp.zeros((16, 64), F32).at[0:10].set(W1[:, 4:14].T)
    net_mids = [(npp[i][0].T, bias(npp[i][1])) for i in (1, 2, 3)]
    net_wl = jnp.zeros((64, 16), F32).at[:, 0:4].set(npp[4][0].T)
    net_bl = jnp.zeros((1, 16), F32).at[0, 0:4].set(npp[4][1])
    res_mask = jnp.zeros((1, 16), F32).at[0, 0:4].set(1.0)

    def fold_base(W):
        wl = jnp.zeros((16, W.shape[0]), F32).at[0:4].set(W[:, 0:4].T)
        ws = jnp.zeros((16, W.shape[0]), F32).at[0:10].set(W[:, 4:14].T)
        wd = jnp.zeros((16, W.shape[0]), F32).at[0:10].set(W[:, 14:24].T)
        return wl, ws, wd

    inv = np.float32(1.0 / np.sqrt(8.0))
    kw = msa['k_w'][:, 0]
    A = (kw[:, None] * msa['q_w']).reshape(4, 8, 32).sum(1) * inv
    c_att = (kw * msa['q_b']).reshape(4, 8).sum(1) * inv
    Wq, bq = lq[4]
    ws_att = jnp.einsum('hd,ldk->lhk', A, Wq.reshape(4, 32, 64)).reshape(16, 64)
    bs_att = (jnp.einsum('hd,ld->lh', A, bq.reshape(4, 32)) + c_att[None, :]).reshape(16)
    vw = msa['v_w'][:, 0]
    M = jnp.einsum('nhj,hj->nh', msa['o_w'].reshape(32, 4, 8), vw.reshape(4, 8))
    o_c = msa['o_w'] @ msa['v_b'] + msa['o_b']

    lq1a, lq1b, lq1c = fold_base(lq[0][0])
    lq_mids = [(lq[i][0].T, bias(lq[i][1])) for i in (1, 2, 3)]
    lq_w = (lq1a, lq1b, lq1c, bias(lq[0][1]), lq_mids, ws_att.T, bias(bs_att))

    Wc1, bc1 = ca[0]
    ca1a, ca1b, ca1c = fold_base(Wc1[:, 0:24])
    Wout = Wc1[:, 24:].reshape(64, 4, 32)
    wt = jnp.einsum('nlq,qh->nlh', Wout, M).reshape(64, 16).T
    bc1e = bias(bc1 + jnp.einsum('nlq,q->n', Wout, o_c))
    ca_mids = [(ca[i][0].T, bias(ca[i][1])) for i in (1, 2, 3)]
    ca_w5 = jnp.zeros((64, 80), F32).at[:, 0:69].set(ca[4][0].T)
    ca_b5 = jnp.zeros((1, 80), F32).at[0, 0:69].set(ca[4][1])
    ca_w = (ca1a, ca1b, ca1c, wt, bc1e, ca_mids, ca_w5, ca_b5)

    Wr1 = cr[0][0]
    wr_a = jnp.zeros((16, 64), F32).at[0:10].set(Wr1[:, 0:10].T)
    wr_b = jnp.zeros((48, 64), F32).at[0:32].set(Wr1[:, 10:42].T)
    wr_c = Wr1[:, 42:74].T
    cr_mids = [(cr[i][0].T, bias(cr[i][1])) for i in (1, 2, 3)]
    cr_wl = jnp.zeros((64, 16), F32).at[:, 0:4].set(cr[4][0].T)
    cr_bl = jnp.zeros((1, 16), F32).at[0, 0:4].set(cr[4][1])

    def bd(w):
        z = jnp.zeros_like(w)
        return jnp.concatenate([jnp.concatenate([w, z], 1),
                                jnp.concatenate([z, w], 1)], 0)

    def t2(b):
        return jnp.concatenate([b, b], 1)

    net_mids_p = [(bd(w), t2(b)) for (w, b) in net_mids]

    return dict(
        w1_src1=bd(w1_src1), w1_dst=bd(w1_dst), w1_l3=bd(w1_l3),
        w1_s3=bd(w1_s3),
        net_mids=net_mids_p, net_wl=bd(net_wl), net_bl=t2(net_bl),
        net_b1=t2(bias(npp[0][1])),
        res_mask=t2(res_mask),
        lq_w=lq_w, ca_w=ca_w,
        wr_a=wr_a, wr_b=wr_b, wr_c=wr_c, cr_mids=cr_mids, cr_wl=cr_wl,
        cr_bl=cr_bl, cr_b1=bias(cr[0][1]),
    )


def _padi(x, n, val):
    return jnp.concatenate([x, jnp.full((n - x.shape[0],), val, I32)])


def kernel(nf, n_atslew, ef_cell, params, edge_index_net1, edge_index_cell,
           edge_index_net3):
    fw = _fold_weights(params)

    nf16 = jnp.pad(nf, ((0, 0), (0, 6)))
    a16 = jnp.pad(jnp.concatenate([n_atslew, nf], axis=1), ((0, 0), (0, 2)))
    efpt = jnp.pad(ef_cell, ((0, E2P - E2), (0, 0))).T

    s1p = _padi(edge_index_net1[0], E1P, 0)
    d1p = _padi(edge_index_net1[1], E1P, 0)
    s2p = _padi(edge_index_cell[0], E2P, 0)
    d2p = _padi(edge_index_cell[1], E2P, 0)
    s3p = _padi(edge_index_net3[0], E3P, 0)
    d3p = _padi(edge_index_net3[1], E3P, 0)
    dl1 = _padi(edge_index_net1[1] - Q4, E1P, ACCR - 1)
    sl2 = _padi(edge_index_cell[0] - Q4, E2P, 0)
    dl2 = _padi(edge_index_cell[1] - 2 * Q4, E2P, ACCS - 1)
    sl3 = _padi(edge_index_net3[0] - 2 * Q4, E3P, 0)
    dl3 = _padi(edge_index_net3[1] - 3 * Q4, E3P, ACCR - 1)

    zeros_acc = jnp.zeros((ACCR, 16), F32)
    zeros_acc48 = jnp.zeros((ACCS, 48), F32)

    g_s1, g_d1 = _make_sc_gather([(CB1, 16), (CB1, 16)])(
        a16, nf16, s1p, d1p)

    efn1 = _run_net_mlp([g_s1.reshape(E1P // 2, 32),
                         g_d1.reshape(E1P // 2, 32)],
                        [fw['w1_src1'], fw['w1_dst']],
                        fw['net_b1'], fw['net_mids'], fw['net_wl'],
                        fw['net_bl'], fw['res_mask'], 1120).reshape(E1P, 16)
    agg1, g_l2, g_s2, g_d2, g_s3, g_d3 = _make_sc_scatter_gather(
        CBS1, CBG2, [E2P // NS, E2P // NS, E3P // NS, E3P // NS])(
        efn1, dl1, sl2, zeros_acc, nf16, s2p, d2p, s3p, d3p)

    efc1c, efc2, efce_p = _run_level2(g_l2, g_s2, g_d2, efpt, fw['lq_w'],
                                      fw['ca_w'], 512)
    sum48, max32 = _make_sc_reduce2()(efc1c, efc2, dl2, zeros_acc48)

    nfq3 = jnp.pad(nf16[2 * Q4:3 * Q4], ((0, 60), (0, 0)))
    m0 = jnp.pad(max32[:OUTR], ((0, 2560 - OUTR), (0, 0)))
    m1 = jnp.pad(max32[OUTR:], ((0, 2560 - OUTR), (0, 0)))
    red16 = _run_cellreduce(nfq3, sum48[:ACCS], sum48[ACCS:], m0, m1,
                            fw['wr_a'], fw['wr_b'], fw['wr_c'], fw['cr_b1'],
                            fw['cr_mids'], fw['cr_wl'], fw['cr_bl'], 2560)

    g_l3 = _make_sc_gather([(CB3, 16)])(red16, sl3)
    efn3 = _run_net_mlp([g_l3.reshape(E3P // 2, 32),
                         g_s3.reshape(E3P // 2, 32),
                         g_d3.reshape(E3P // 2, 32)],
                        [fw['w1_l3'], fw['w1_s3'], fw['w1_dst']],
                        fw['net_b1'], fw['net_mids'], fw['net_wl'],
                        fw['net_bl'], fw['res_mask'], 880).reshape(E3P, 16)
    agg3 = _make_sc_scatter_single(CBS3)(efn3, dl3, zeros_acc)

    new_nf = jnp.concatenate([
        n_atslew[:Q4],
        agg1[:Q4, 0:4],
        red16[:Q4, 0:4],
        agg3[:Q4, 0:4],
    ], axis=0)
    return new_nf, efce_p[:E2]


# speedup vs baseline: 1.0190x; 1.0190x over previous
import functools

import jax
import jax.numpy as jnp
import numpy as np
from jax import lax
from jax.experimental import pallas as pl
from jax.experimental.pallas import tpu as pltpu
from jax.experimental.pallas import tpu_sc as plsc

F32 = jnp.float32
I32 = jnp.int32

NC, NS, NW = 2, 16, 32
N = 10000
Q4 = N // 4
E1, E2, E3 = 80000, 10000, 70000
E1P, E2P, E3P = 80640, 10240, 70400
CB1, CB2, CB3 = E1P // NW, E2P // NW, E3P // NW
CBS1, CBS3 = E1P // NS, E3P // NS
CBG2, CBSC2 = E2P // NS, E2P // NS
ACCR = 2504
ACCS = 2560
RPW = 79
OUTR = RPW * NW
BATCH = 512

L = 4
SZ = 7
AXIS_LEN = L * (1 + 2 * SZ)


def _mesh():
    return plsc.VectorSubcoreMesh(core_axis_name="c", subcore_axis_name="s")


_SC_PARAMS = pltpu.CompilerParams(use_tc_tiling_on_sc=False,
                                  needs_layout_passes=False)


def _make_sc_gather(specs):
    n = len(specs)
    cbmax = max(cb for cb, _ in specs)
    wmax = max(w for _, w in specs)
    out_type = tuple(jax.ShapeDtypeStruct((cb * NW, w), F32) for cb, w in specs)
    if n == 1:
        out_type = out_type[0]
    scratch = [pltpu.VMEM((cbmax,), I32), pltpu.VMEM((cbmax, wmax), F32),
               pltpu.SemaphoreType.DMA]

    @functools.partial(pl.kernel, mesh=_mesh(), out_type=out_type,
                       scratch_types=scratch, compiler_params=_SC_PARAMS)
    def k(*refs):
        tables = refs[:n]
        idxs = refs[n:2 * n]
        outs = refs[2 * n:3 * n]
        idx_v, rows_v, sem = refs[3 * n:3 * n + 3]
        wid = lax.axis_index("s") * NC + lax.axis_index("c")
        for g, (cb, w) in enumerate(specs):
            base = wid * cb
            iv = idx_v.at[pl.ds(0, cb)]
            rv = rows_v.at[pl.ds(0, cb), pl.ds(0, w)]
            pltpu.sync_copy(idxs[g].at[pl.ds(base, cb)], iv)
            pltpu.async_copy(tables[g].at[iv], rv, sem).wait()
            pltpu.sync_copy(rv, outs[g].at[pl.ds(base, cb)])

    return k


def _make_sc_scatter_gather(cbs, cbg, side_specs):
    cbmax = max([cbs] + list(side_specs))
    out_type = (jax.ShapeDtypeStruct((ACCR, 16), F32),
                jax.ShapeDtypeStruct((cbg * NS, 16), F32)) + tuple(
        jax.ShapeDtypeStruct((cb * NS, 16), F32) for cb in side_specs)
    nside = len(side_specs)
    scratch = [pltpu.VMEM((cbmax, 16), F32), pltpu.VMEM((cbmax,), I32),
               pltpu.VMEM((cbg,), I32), pltpu.VMEM((cbg, 16), F32),
               pltpu.VMEM_SHARED((ACCR, 16), F32), pltpu.SemaphoreType.DMA]

    @functools.partial(pl.kernel, mesh=_mesh(), out_type=out_type,
                       scratch_types=scratch, compiler_params=_SC_PARAMS)
    def k(*refs):
        (vals_hbm, dstl_hbm, gidx_hbm, zeros_hbm, table_hbm) = refs[:5]
        sidx_hbm = refs[5:5 + nside]
        acc_out, g_out = refs[5 + nside], refs[6 + nside]
        souts = refs[7 + nside:7 + 2 * nside]
        vals_v, idx_v, gi_v, gr_v, acc_sh, sem = refs[7 + 2 * nside:]
        cid = lax.axis_index("c")
        sid = lax.axis_index("s")

        @pl.when(cid == 0)
        def _():
            @pl.when(sid == 0)
            def _():
                pltpu.sync_copy(zeros_hbm, acc_sh)

            plsc.subcore_barrier()
            base = sid * cbs
            pltpu.sync_copy(vals_hbm.at[pl.ds(base, cbs)],
                            vals_v.at[pl.ds(0, cbs), :])
            pltpu.sync_copy(dstl_hbm.at[pl.ds(base, cbs)],
                            idx_v.at[pl.ds(0, cbs)])
            pltpu.sync_copy(vals_v.at[pl.ds(0, cbs), :],
                            acc_sh.at[idx_v.at[pl.ds(0, cbs)]], add=True)
            plsc.subcore_barrier()
            gbase = sid * cbg
            pltpu.sync_copy(gidx_hbm.at[pl.ds(gbase, cbg)], gi_v)
            pltpu.async_copy(acc_sh.at[gi_v], gr_v, sem).wait()
            pltpu.sync_copy(gr_v, g_out.at[pl.ds(gbase, cbg)])

            @pl.when(sid == 0)
            def _():
                pltpu.sync_copy(acc_sh, acc_out)

        @pl.when(cid == 1)
        def _():
            for g in range(nside):
                cb = side_specs[g]
                gb = sid * cb
                iv = idx_v.at[pl.ds(0, cb)]
                rv = vals_v.at[pl.ds(0, cb), :]
                pltpu.sync_copy(sidx_hbm[g].at[pl.ds(gb, cb)], iv)
                pltpu.async_copy(table_hbm.at[iv], rv, sem).wait()
                pltpu.sync_copy(rv, souts[g].at[pl.ds(gb, cb)])

    return k


def _make_sc_scatter_single(cbs):
    scratch = [pltpu.VMEM((cbs, 16), F32), pltpu.VMEM((cbs,), I32),
               pltpu.VMEM_SHARED((ACCR, 16), F32)]

    @functools.partial(pl.kernel, mesh=_mesh(),
                       out_type=jax.ShapeDtypeStruct((ACCR, 16), F32),
                       scratch_types=scratch, compiler_params=_SC_PARAMS)
    def k(vals_hbm, dstl_hbm, zeros_hbm, acc_out, vals_v, idx_v, acc_sh):
        cid = lax.axis_index("c")
        sid = lax.axis_index("s")

        @pl.when(cid == 0)
        def _():
            @pl.when(sid == 0)
            def _():
                pltpu.sync_copy(zeros_hbm, acc_sh)

            plsc.subcore_barrier()
            base = sid * cbs
            pltpu.sync_copy(vals_hbm.at[pl.ds(base, cbs)], vals_v)
            pltpu.sync_copy(dstl_hbm.at[pl.ds(base, cbs)], idx_v)
            pltpu.sync_copy(vals_v, acc_sh.at[idx_v], add=True)
            plsc.subcore_barrier()

            @pl.when(sid == 0)
            def _():
                pltpu.sync_copy(acc_sh, acc_out)

    return k


def _make_sc_reduce2():
    EH = E2P // 2
    CBH = EH // NS
    NVEC = EH // 16
    RPW2 = OUTR // NS
    scratch = [
        pltpu.VMEM((EH,), I32),
        pltpu.VMEM((EH + BATCH + 16,), I32),
        pltpu.VMEM((EH + 16,), I32),
        pltpu.VMEM((BATCH, 32), F32),
        pltpu.VMEM((RPW2, 32), F32),
        pltpu.VMEM((CBH, 48), F32),
        pltpu.VMEM((CBH,), I32),
        pltpu.VMEM_SHARED((ACCS, 48), F32),
        pltpu.SemaphoreType.DMA,
    ]

    @functools.partial(pl.kernel, mesh=_mesh(),
                       out_type=(jax.ShapeDtypeStruct((NC * ACCS, 48), F32),
                                 jax.ShapeDtypeStruct((NC * OUTR, 32), F32)),
                       scratch_types=scratch, compiler_params=_SC_PARAMS)
    def k(efc1_hbm, efc2_hbm, dstl_hbm, zeros_hbm, sum_out, max_out,
          dst_v, hit_ids, hit_dst, rows_v, acc, svals, sidx, acc_sh, sem):
        cid = lax.axis_index("c")
        sid = lax.axis_index("s")
        ebase = cid * EH

        @pl.when(sid == 0)
        def _():
            pltpu.sync_copy(zeros_hbm, acc_sh)

        plsc.subcore_barrier()
        base = ebase + sid * CBH
        pltpu.sync_copy(efc1_hbm.at[pl.ds(base, CBH)], svals)
        pltpu.sync_copy(dstl_hbm.at[pl.ds(base, CBH)], sidx)
        pltpu.sync_copy(svals, acc_sh.at[sidx], add=True)
        plsc.subcore_barrier()

        @pl.when(sid == 0)
        def _():
            pltpu.sync_copy(acc_sh, sum_out.at[pl.ds(cid * ACCS, ACCS)])

        lo = sid * RPW2
        hi = lo + RPW2
        pltpu.sync_copy(dstl_hbm.at[pl.ds(ebase, EH)], dst_v)

        ninf = jnp.full((16,), -3.4e38, F32)
        zi = jnp.zeros((16,), I32)

        def init_r(r, c):
            acc[r, pl.ds(0, 16)] = ninf
            acc[r, pl.ds(16, 16)] = ninf
            return c

        lax.fori_loop(0, RPW2, init_r, 0)

        iota16 = lax.iota(I32, 16)

        def scan_b(v, off_v):
            dv = dst_v[pl.ds(v * 16, 16)]
            m = (dv >= lo) & (dv < hi)
            pc = plsc.cumsum(m.astype(I32))
            pos = off_v + pc - 1
            plsc.store_scatter(hit_ids, [pos], ebase + iota16 + v * 16, mask=m)
            plsc.store_scatter(hit_dst, [pos], dv, mask=m)
            return off_v + plsc.all_reduce_population_count(m)

        off_v = lax.fori_loop(0, NVEC, scan_b, jnp.zeros((16,), I32))
        nh = off_v[0]

        def zt(i, c):
            hit_ids[pl.ds(nh + i * 16, 16)] = zi
            return c

        lax.fori_loop(0, BATCH // 16 + 1, zt, 0)

        nbt = (nh + BATCH - 1) // BATCH

        def batch_b(b, c):
            ids = hit_ids.at[pl.ds(b * BATCH, BATCH)]
            pltpu.async_copy(efc2_hbm.at[ids], rows_v, sem).wait()
            nb = jnp.minimum(BATCH, nh - b * BATCH)

            def e_b(j, cc):
                dvec = hit_dst[pl.ds(b * BATCH + j, 16)]
                r = dvec[0] - lo
                acc[r, pl.ds(0, 16)] = jnp.maximum(acc[r, pl.ds(0, 16)],
                                                   rows_v[j, pl.ds(0, 16)])
                acc[r, pl.ds(16, 16)] = jnp.maximum(acc[r, pl.ds(16, 16)],
                                                    rows_v[j, pl.ds(16, 16)])
                return cc

            lax.fori_loop(0, nb, e_b, 0)
            return c

        lax.fori_loop(0, nbt, batch_b, 0)
        pltpu.sync_copy(acc, max_out.at[pl.ds(cid * OUTR + lo, RPW2)])

    return k


def _run_net_mlp(parts, w1_parts, b1, mids, wlast, blast, res_mask, be):
    ep = parts[0].shape[0]
    nblk = ep // be
    npart = len(parts)

    def body(*refs):
        ins = refs[:npart]
        i = npart
        w1r = refs[i:i + npart]; i += npart
        b1r = refs[i]; i += 1
        midr = refs[i:i + 6]; i += 6
        wlr, blr, mr = refs[i], refs[i + 1], refs[i + 2]
        out = refs[-1]
        x0 = ins[0][...]
        s = jnp.dot(x0, w1r[0][...], preferred_element_type=F32)
        for p in range(1, npart):
            s = s + jnp.dot(ins[p][...], w1r[p][...], preferred_element_type=F32)
        h = jax.nn.relu(s + b1r[...])
        for m in range(3):
            h = jax.nn.relu(jnp.dot(h, midr[2 * m][...],
                                    preferred_element_type=F32) + midr[2 * m + 1][...])
        y = jnp.dot(h, wlr[...], preferred_element_type=F32) + blr[...]
        out[...] = y + x0 * mr[...]

    in_specs = [pl.BlockSpec((be, p.shape[1]), lambda i: (i, 0)) for p in parts]
    wargs = list(w1_parts) + [b1]
    for (w, b) in mids:
        wargs += [w, b]
    wargs += [wlast, blast, res_mask]
    in_specs += [pl.BlockSpec(w.shape, lambda i: (0, 0)) for w in wargs]
    return pl.pallas_call(
        body,
        grid=(nblk,),
        in_specs=in_specs,
        out_specs=pl.BlockSpec((be, wlast.shape[1]), lambda i: (i, 0)),
        out_shape=jax.ShapeDtypeStruct((ep, wlast.shape[1]), F32),
    )(*parts, *wargs)


def _run_level2(g_l2, g_s2, g_d2, efp, lq_w, ca_w, be):
    nblk = E2P // be
    lq_flat = [lq_w[0], lq_w[1], lq_w[2], lq_w[3]]
    for (w, b) in lq_w[4]:
        lq_flat += [w, b]
    lq_flat += [lq_w[5], lq_w[6]]
    ca_flat = [ca_w[0], ca_w[1], ca_w[2], ca_w[3], ca_w[4]]
    for (w, b) in ca_w[5]:
        ca_flat += [w, b]
    ca_flat += [ca_w[6], ca_w[7]]
    nlq = len(lq_flat)

    def body(*refs):
        gl, gs, gd, eft = (r[...] for r in refs[:4])
        i = 4
        lw = refs[i:i + nlq]; i += nlq
        cw = refs[i:i + len(ca_flat)]; i += len(ca_flat)
        o1c, o2, oce = refs[-3], refs[-2], refs[-1]

        def mm(a, b):
            return jnp.dot(a, b, preferred_element_type=F32)

        h = jax.nn.relu(mm(gl, lw[0][...]) + mm(gs, lw[1][...]) +
                        mm(gd, lw[2][...]) + lw[3][...])
        for m in range(3):
            h = jax.nn.relu(mm(h, lw[4 + 2 * m][...]) + lw[5 + 2 * m][...])
        s = mm(h, lw[10][...]) + lw[11][...]
        st = jnp.transpose(s)

        trows = []
        for l in range(L):
            k1t = eft[15 * l + 1:15 * l + 8, :]
            k2t = eft[15 * l + 8:15 * l + 15, :]
            tabt = eft[AXIS_LEN + 49 * l:AXIS_LEN + 49 * l + 49, :]
            kmt = jnp.concatenate([k1t[i2:i2 + 1, :] * k2t for i2 in range(SZ)],
                                  axis=0)
            for hh in range(4):
                lg = st[4 * l + hh:4 * l + hh + 1, :] * kmt
                mx = jnp.max(lg, axis=0, keepdims=True)
                p = jnp.exp(lg - mx)
                z = jnp.sum(p, axis=0, keepdims=True)
                trows.append(jnp.sum(p * tabt, axis=0, keepdims=True) / z)
        t = jnp.transpose(jnp.concatenate(trows, axis=0))

        h2 = jax.nn.relu(mm(gl, cw[0][...]) + mm(gs, cw[1][...]) +
                         mm(gd, cw[2][...]) + mm(t, cw[3][...]) + cw[4][...])
        for m in range(3):
            h2 = jax.nn.relu(mm(h2, cw[5 + 2 * m][...]) + cw[6 + 2 * m][...])
        x80 = mm(h2, cw[11][...]) + cw[12][...]
        kk = jax.nn.sigmoid(x80[:, 0:1])
        ones = jnp.ones((x80.shape[0], 1), F32)
        zer = jnp.zeros((x80.shape[0], 15), F32)
        o1c[...] = jnp.concatenate([x80[:, 1:33] * kk, ones, zer], axis=1)
        o2[...] = x80[:, 33:65] * kk
        oce[...] = x80[:, 65:69]

    in_specs = [
        pl.BlockSpec((be, 16), lambda i: (i, 0)),
        pl.BlockSpec((be, 16), lambda i: (i, 0)),
        pl.BlockSpec((be, 16), lambda i: (i, 0)),
        pl.BlockSpec((256, be), lambda i: (0, i)),
    ]
    wargs = lq_flat + ca_flat
    in_specs += [pl.BlockSpec(w.shape, lambda i: (0, 0)) for w in wargs]
    return pl.pallas_call(
        body,
        grid=(nblk,),
        in_specs=in_specs,
        out_specs=[pl.BlockSpec((be, 48), lambda i: (i, 0)),
                   pl.BlockSpec((be, 32), lambda i: (i, 0)),
                   pl.BlockSpec((be, 4), lambda i: (i, 0))],
        out_shape=[jax.ShapeDtypeStruct((E2P, 48), F32),
                   jax.ShapeDtypeStruct((E2P, 32), F32),
                   jax.ShapeDtypeStruct((E2P, 4), F32)],
    )(g_l2, g_s2, g_d2, efp, *wargs)


def _run_cellreduce(nfq3, s0, s1, m0, m1, wa, wb, wc, b1, mids, wlast, blast,
                    be):
    ep = nfq3.shape[0]
    nblk = ep // be
    wargs = [wa, wb, wc, b1]
    for (w, b) in mids:
        wargs += [w, b]
    wargs += [wlast, blast]

    def body(*refs):
        xn, xs0, xs1, xm0, xm1 = (r[...] for r in refs[:5])
        w = refs[5:-1]
        out = refs[-1]
        xs = xs0 + xs1
        xm = jnp.maximum(xm0, xm1)
        cnt = xs[:, 32:33]
        fixed = jnp.where(((xm - xm) == 0.0) & (cnt > 0.0), xm, 0.0)
        h = jax.nn.relu(jnp.dot(xn, w[0][...], preferred_element_type=F32) +
                        jnp.dot(xs, w[1][...], preferred_element_type=F32) +
                        jnp.dot(fixed, w[2][...], preferred_element_type=F32) +
                        w[3][...])
        for m in range(3):
            h = jax.nn.relu(jnp.dot(h, w[4 + 2 * m][...],
                                    preferred_element_type=F32) + w[5 + 2 * m][...])
        out[...] = jnp.dot(h, w[10][...], preferred_element_type=F32) + w[11][...]

    in_specs = [pl.BlockSpec((be, 16), lambda i: (i, 0)),
                pl.BlockSpec((be, 48), lambda i: (i, 0)),
                pl.BlockSpec((be, 48), lambda i: (i, 0)),
                pl.BlockSpec((be, 32), lambda i: (i, 0)),
                pl.BlockSpec((be, 32), lambda i: (i, 0))]
    in_specs += [pl.BlockSpec(w.shape, lambda i: (0, 0)) for w in wargs]
    return pl.pallas_call(
        body,
        grid=(nblk,),
        in_specs=in_specs,
        out_specs=pl.BlockSpec((be, 16), lambda i: (i, 0)),
        out_shape=jax.ShapeDtypeStruct((ep, 16), F32),
    )(nfq3, s0, s1, m0, m1, *wargs)


def _fold_weights(params):
    npp = params['netprop']
    lq = params['lut_query']
    ca = params['cellarc']
    cr = params['cellreduce']
    msa = params['msa']

    def bias(b):
        return b.reshape(1, -1)

    W1 = npp[0][0]
    w1_src1 = jnp.zeros((16, 64), F32).at[0:4].set(W1[:, 0:4].T).at[4:14].set(W1[:, 4:14].T)
    w1_dst = jnp.zeros((16, 64), F32).at[0:10].set(W1[:, 14:24].T)
    w1_l3 = jnp.zeros((16, 64), F32).at[0:4].set(W1[:, 0:4].T)
    w1_s3 = jnp.zeros((16, 64), F32).at[0:10].set(W1[:, 4:14].T)
    net_mids = [(npp[i][0].T, bias(npp[i][1])) for i in (1, 2, 3)]
    net_wl = jnp.zeros((64, 16), F32).at[:, 0:4].set(npp[4][0].T)
    net_bl = jnp.zeros((1, 16), F32).at[0, 0:4].set(npp[4][1])
    res_mask = jnp.zeros((1, 16), F32).at[0, 0:4].set(1.0)

    def fold_base(W):
        wl = jnp.zeros((16, W.shape[0]), F32).at[0:4].set(W[:, 0:4].T)
        ws = jnp.zeros((16, W.shape[0]), F32).at[0:10].set(W[:, 4:14].T)
        wd = jnp.zeros((16, W.shape[0]), F32).at[0:10].set(W[:, 14:24].T)
        return wl, ws, wd

    inv = np.float32(1.0 / np.sqrt(8.0))
    kw = msa['k_w'][:, 0]
    A = (kw[:, None] * msa['q_w']).reshape(4, 8, 32).sum(1) * inv
    c_att = (kw * msa['q_b']).reshape(4, 8).sum(1) * inv
    Wq, bq = lq[4]
    ws_att = jnp.einsum('hd,ldk->lhk', A, Wq.reshape(4, 32, 64)).reshape(16, 64)
    bs_att = (jnp.einsum('hd,ld->lh', A, bq.reshape(4, 32)) + c_att[None, :]).reshape(16)
    vw = msa['v_w'][:, 0]
    M = jnp.einsum('nhj,hj->nh', msa['o_w'].reshape(32, 4, 8), vw.reshape(4, 8))
    o_c = msa['o_w'] @ msa['v_b'] + msa['o_b']

    lq1a, lq1b, lq1c = fold_base(lq[0][0])
    lq_mids = [(lq[i][0].T, bias(lq[i][1])) for i in (1, 2, 3)]
    lq_w = (lq1a, lq1b, lq1c, bias(lq[0][1]), lq_mids, ws_att.T, bias(bs_att))

    Wc1, bc1 = ca[0]
    ca1a, ca1b, ca1c = fold_base(Wc1[:, 0:24])
    Wout = Wc1[:, 24:].reshape(64, 4, 32)
    wt = jnp.einsum('nlq,qh->nlh', Wout, M).reshape(64, 16).T
    bc1e = bias(bc1 + jnp.einsum('nlq,q->n', Wout, o_c))
    ca_mids = [(ca[i][0].T, bias(ca[i][1])) for i in (1, 2, 3)]
    ca_w5 = jnp.zeros((64, 80), F32).at[:, 0:69].set(ca[4][0].T)
    ca_b5 = jnp.zeros((1, 80), F32).at[0, 0:69].set(ca[4][1])
    ca_w = (ca1a, ca1b, ca1c, wt, bc1e, ca_mids, ca_w5, ca_b5)

    Wr1 = cr[0][0]
    wr_a = jnp.zeros((16, 64), F32).at[0:10].set(Wr1[:, 0:10].T)
    wr_b = jnp.zeros((48, 64), F32).at[0:32].set(Wr1[:, 10:42].T)
    wr_c = Wr1[:, 42:74].T
    cr_mids = [(cr[i][0].T, bias(cr[i][1])) for i in (1, 2, 3)]
    cr_wl = jnp.zeros((64, 16), F32).at[:, 0:4].set(cr[4][0].T)
    cr_bl = jnp.zeros((1, 16), F32).at[0, 0:4].set(cr[4][1])

    def bd(w):
        z = jnp.zeros_like(w)
        return jnp.concatenate([jnp.concatenate([w, z], 1),
                                jnp.concatenate([z, w], 1)], 0)

    def t2(b):
        return jnp.concatenate([b, b], 1)

    net_mids_p = [(bd(w), t2(b)) for (w, b) in net_mids]

    return dict(
        w1_src1=bd(w1_src1), w1_dst=bd(w1_dst), w1_l3=bd(w1_l3),
        w1_s3=bd(w1_s3),
        net_mids=net_mids_p, net_wl=bd(net_wl), net_bl=t2(net_bl),
        net_b1=t2(bias(npp[0][1])),
        res_mask=t2(res_mask),
        lq_w=lq_w, ca_w=ca_w,
        wr_a=wr_a, wr_b=wr_b, wr_c=wr_c, cr_mids=cr_mids, cr_wl=cr_wl,
        cr_bl=cr_bl, cr_b1=bias(cr[0][1]),
    )


def _padi(x, n, val):
    return jnp.concatenate([x, jnp.full((n - x.shape[0],), val, I32)])


def kernel(nf, n_atslew, ef_cell, params, edge_index_net1, edge_index_cell,
           edge_index_net3):
    fw = _fold_weights(params)

    nf16 = jnp.pad(nf, ((0, 0), (0, 6)))
    a16 = jnp.pad(jnp.concatenate([n_atslew, nf], axis=1), ((0, 0), (0, 2)))
    efpt = ef_cell.T

    s1p = _padi(edge_index_net1[0], E1P, 0)
    d1p = _padi(edge_index_net1[1], E1P, 0)
    s2p = _padi(edge_index_cell[0], E2P, 0)
    d2p = _padi(edge_index_cell[1], E2P, 0)
    s3p = _padi(edge_index_net3[0], E3P, 0)
    d3p = _padi(edge_index_net3[1], E3P, 0)
    dl1 = _padi(edge_index_net1[1] - Q4, E1P, ACCR - 1)
    sl2 = _padi(edge_index_cell[0] - Q4, E2P, 0)
    dl2 = _padi(edge_index_cell[1] - 2 * Q4, E2P, ACCS - 1)
    sl3 = _padi(edge_index_net3[0] - 2 * Q4, E3P, 0)
    dl3 = _padi(edge_index_net3[1] - 3 * Q4, E3P, ACCR - 1)

    zeros_acc = jnp.zeros((ACCR, 16), F32)
    zeros_acc48 = jnp.zeros((ACCS, 48), F32)

    g_s1, g_d1 = _make_sc_gather([(CB1, 16), (CB1, 16)])(
        a16, nf16, s1p, d1p)

    efn1 = _run_net_mlp([g_s1.reshape(E1P // 2, 32),
                         g_d1.reshape(E1P // 2, 32)],
                        [fw['w1_src1'], fw['w1_dst']],
                        fw['net_b1'], fw['net_mids'], fw['net_wl'],
                        fw['net_bl'], fw['res_mask'], 1120).reshape(E1P, 16)
    agg1, g_l2, g_s2, g_d2, g_s3, g_d3 = _make_sc_scatter_gather(
        CBS1, CBG2, [E2P // NS, E2P // NS, E3P // NS, E3P // NS])(
        efn1, dl1, sl2, zeros_acc, nf16, s2p, d2p, s3p, d3p)

    efc1c, efc2, efce_p = _run_level2(g_l2, g_s2, g_d2, efpt, fw['lq_w'],
                                      fw['ca_w'], 512)
    sum48, max32 = _make_sc_reduce2()(efc1c, efc2, dl2, zeros_acc48)

    nfq3 = jnp.pad(nf16[2 * Q4:3 * Q4], ((0, 60), (0, 0)))
    m0 = jnp.pad(max32[:OUTR], ((0, 2560 - OUTR), (0, 0)))
    m1 = jnp.pad(max32[OUTR:], ((0, 2560 - OUTR), (0, 0)))
    red16 = _run_cellreduce(nfq3, sum48[:ACCS], sum48[ACCS:], m0, m1,
                            fw['wr_a'], fw['wr_b'], fw['wr_c'], fw['cr_b1'],
                            fw['cr_mids'], fw['cr_wl'], fw['cr_bl'], 2560)

    g_l3 = _make_sc_gather([(CB3, 16)])(red16, sl3)
    efn3 = _run_net_mlp([g_l3.reshape(E3P // 2, 32),
                         g_s3.reshape(E3P // 2, 32),
                         g_d3.reshape(E3P // 2, 32)],
                        [fw['w1_l3'], fw['w1_s3'], fw['w1_dst']],
                        fw['net_b1'], fw['net_mids'], fw['net_wl'],
                        fw['net_bl'], fw['res_mask'], 880).reshape(E3P, 16)
    agg3 = _make_sc_scatter_single(CBS3)(efn3, dl3, zeros_acc)

    new_nf = jnp.concatenate([
        n_atslew[:Q4],
        agg1[:Q4, 0:4],
        red16[:Q4, 0:4],
        agg3[:Q4, 0:4],
    ], axis=0)
    return new_nf, efce_p[:E2]
